# baseline (speedup 1.0000x reference)
import jax
import jax.numpy as jnp
from jax.experimental import pallas as pl
from jax.experimental.pallas import tpu as pltpu

K = 20
C = 256
BN = 64


def _tc_body(fts_ref, knn_ref, out_ref):
    c = fts_ref[0]
    k = knn_ref[0]
    cb = c[:, None, :]
    out_ref[0, :, :, :C] = jnp.broadcast_to(cb, (BN, K, C))
    out_ref[0, :, :, C:] = k - cb


def kernel(fts, knn_fts):
    B, N, _ = fts.shape
    grid = (B, N // BN)
    out = pl.pallas_call(
        _tc_body,
        grid=grid,
        in_specs=[
            pl.BlockSpec((1, BN, C), lambda b, i: (b, i, 0)),
            pl.BlockSpec((1, BN, K, C), lambda b, i: (b, i, 0, 0)),
        ],
        out_specs=pl.BlockSpec((1, BN, K, 2 * C), lambda b, i: (b, i, 0, 0)),
        out_shape=jax.ShapeDtypeStruct((B, N, K, 2 * C), fts.dtype),
    )(fts, knn_fts)
    return out


# speedup vs baseline: 3.3149x; 3.3149x over previous
import jax
import jax.numpy as jnp
from jax.experimental import pallas as pl

K = 20
C = 256


def _tc_body(fts_ref, knn_ref, out_ref):
    c = fts_ref[0]
    out_ref[0, 0, :, :C] = c
    out_ref[0, 0, :, C:] = knn_ref[0, 0] - c


def kernel(fts, knn_fts):
    B, N, _ = fts.shape
    knn_t = jnp.transpose(knn_fts, (0, 2, 1, 3))
    out_t = pl.pallas_call(
        _tc_body,
        grid=(B, K),
        in_specs=[
            pl.BlockSpec((1, N, C), lambda b, k: (b, 0, 0)),
            pl.BlockSpec((1, 1, N, C), lambda b, k: (b, k, 0, 0)),
        ],
        out_specs=pl.BlockSpec((1, 1, N, 2 * C), lambda b, k: (b, k, 0, 0)),
        out_shape=jax.ShapeDtypeStruct((B, K, N, 2 * C), fts.dtype),
    )(fts, knn_t)
    return jnp.transpose(out_t, (0, 2, 1, 3))
